# baseline (speedup 1.0000x reference)
import functools

import jax
import jax.numpy as jnp
from jax import lax
from jax.experimental import pallas as pl
from jax.experimental.pallas import tpu as pltpu
from jax.experimental.pallas import tpu_sc as plsc

_N = 10000
_E = 320000
_DE = 16
_DIN = 128
_H = 128

_CHUNK = 128
_NCHUNKS = _E // _CHUNK
_NC = 2
_NS = 16
_CPT = 8
_HALF = _NCHUNKS // _NC
_TRIPS = _HALF // _CPT
_TAIL = _HALF - _TRIPS * _CPT
_ROWS_PER_TILE = 632
_NPAD = _ROWS_PER_TILE * _NS
_FOLD = 640


def _sc_segment_sum(ei3, attr4):
    mesh = plsc.VectorSubcoreMesh(core_axis_name="c", subcore_axis_name="s")

    @functools.partial(
        pl.kernel,
        mesh=mesh,
        compiler_params=pltpu.CompilerParams(use_tc_tiling_on_sc=False,
                                             needs_layout_passes=False),
        out_type=[
            jax.ShapeDtypeStruct((_NC, _NS, _NPAD), jnp.float32),
            jax.ShapeDtypeStruct((_NC, _NPAD, _DE), jnp.float32),
        ],
        scratch_types=[
            pltpu.VMEM((2, _CPT, 2, _CHUNK), jnp.int32),
            pltpu.VMEM((2, _CPT, _CHUNK), jnp.float32),
            pltpu.VMEM((4, _NPAD), jnp.float32),
            pltpu.VMEM((_ROWS_PER_TILE, _DE), jnp.float32),
            pltpu.VMEM((_FOLD, _DE), jnp.float32),
            pltpu.VMEM((_FOLD, _DE), jnp.float32),
            pltpu.VMEM((5, _CHUNK), jnp.int32),
            pltpu.VMEM_SHARED((_FOLD, _DE), jnp.float32),
            pltpu.SemaphoreType.DMA((2,)),
            pltpu.SemaphoreType.DMA((2,)),
        ],
    )
    def seg(ei_hbm, attr_hbm, sums_hbm, cnts_hbm,
            idx_v, feat_v, plane_v, stage_v, hist_v, fold_v, iden_v,
            acc_fold, isem, asem):
        cid = lax.axis_index("c")
        sid = lax.axis_index("s")
        ones16 = jnp.ones((_DE,), jnp.float32)
        zeros16 = jnp.zeros((_DE,), jnp.float32)

        def zero_plane(i, carry):
            for p in range(4):
                plane_v[p, pl.ds(i * _DE, _DE)] = zeros16
            return carry
        lax.fori_loop(0, _NPAD // _DE, zero_plane, None)

        def zero_hist(i, carry):
            hist_v[i, :] = zeros16
            return carry
        lax.fori_loop(0, _FOLD, zero_hist, None)

        lane = lax.broadcasted_iota(jnp.int32, (_DE,), 0)
        for c in range(5):
            for k in range(8):
                iden_v[c, pl.ds(k * _DE, _DE)] = lane + (c * _CHUNK + k * _DE)

        pltpu.sync_copy(hist_v.at[pl.ds(sid * (_FOLD // _NS), _FOLD // _NS)],
                        acc_fold.at[pl.ds(sid * (_FOLD // _NS), _FOLD // _NS)])
        plsc.subcore_barrier()

        half0 = cid * _HALF
        fa = sid // 8
        fr = sid % 8

        def loads(t, b, n=_CPT):
            c0 = half0 + t * _CPT
            i_cp = pltpu.make_async_copy(
                ei_hbm.at[pl.ds(c0, n)], idx_v.at[b, pl.ds(0, n)], isem.at[b])
            a_cp = pltpu.make_async_copy(
                attr_hbm.at[fa, pl.ds(c0, n), fr],
                feat_v.at[b, pl.ds(0, n)], asem.at[b])
            return i_cp, a_cp

        def fire(t, b, n=_CPT):
            i_cp, a_cp = loads(t, b, n)
            i_cp.start()
            a_cp.start()

        def consume(t, b, n_chunks):
            i_cp, a_cp = loads(t, b, n_chunks)
            i_cp.wait()
            a_cp.wait()
            for j in range(n_chunks):
                cc = half0 + t * _CPT + j

                ivs = [idx_v[b, j, 1, pl.ds(k * _DE, _DE)]
                       for k in range(8)]
                avs = [feat_v[b, j, pl.ds(k * _DE, _DE)]
                       for k in range(8)]
                for k in range(8):
                    plsc.addupdate_scatter(plane_v.at[k % 4], [ivs[k]],
                                           avs[k])

                @pl.when(cc % _NS == sid)
                def _():
                    for k in range(8):
                        iv = idx_v[b, j, 1, pl.ds(k * _DE, _DE)]
                        plsc.addupdate_scatter(
                            hist_v,
                            [jax.lax.shift_right_logical(iv, 4),
                             jnp.bitwise_and(iv, 15)],
                            ones16)

        fire(0, 0)
        fire(1, 1)

        def body(i, carry):
            for b in range(2):
                t = 2 * i + b
                consume(t, b, _CPT)

                @pl.when(t + 2 < _TRIPS)
                def _():
                    fire(t + 2, b)
            return carry
        lax.fori_loop(0, _TRIPS // 2, body, None)

        fire(_TRIPS, 0, _TAIL)
        consume(_TRIPS, 0, _TAIL)

        def combine(i, carry):
            sl = pl.ds(i * _DE, _DE)
            plane_v[0, sl] = ((plane_v[0, sl] + plane_v[1, sl])
                              + (plane_v[2, sl] + plane_v[3, sl]))
            return carry
        lax.fori_loop(0, _NPAD // _DE, combine, None)
        pltpu.sync_copy(plane_v.at[0], sums_hbm.at[cid, sid])

        for c in range(5):
            pltpu.sync_copy(hist_v.at[pl.ds(c * _CHUNK, _CHUNK)],
                            acc_fold.at[iden_v.at[c]], add=True)

        plsc.subcore_barrier()

        pltpu.sync_copy(acc_fold, fold_v)
        row0 = sid * _ROWS_PER_TILE

        def unfold(n, carry):
            node = row0 + n
            rr = jnp.full((_DE,), jax.lax.shift_right_logical(node, 4),
                          jnp.int32)
            ll = jnp.full((_DE,), jnp.bitwise_and(node, 15), jnp.int32)
            stage_v[n, :] = plsc.load_gather(fold_v, [rr, ll])
            return carry
        lax.fori_loop(0, _ROWS_PER_TILE, unfold, None)
        pltpu.sync_copy(stage_v, cnts_hbm.at[cid, pl.ds(row0, _ROWS_PER_TILE)])

    return seg(ei3, attr4)


_BLK = 1024


def _tc_body(h_ref, w1_ref, w2_ref, s_ref, c_ref, o_ref):
    st = s_ref[0] + s_ref[1]
    cnt = c_ref[0] + c_ref[1]
    proj = lax.dot_general(st, w2_ref[...], (((0,), (0,)), ((), ())),
                           preferred_element_type=jnp.float32)
    recip = 1.0 / jnp.maximum(cnt[:, 0:1], 1.0)
    o_ref[...] = (
        jnp.dot(h_ref[...], w1_ref[...],
                preferred_element_type=jnp.float32)
        + proj * recip
    )


def _tc_combine(h_in, w1t, w2t, sums_t, cnts):
    return pl.pallas_call(
        _tc_body,
        grid=((_N + _BLK - 1) // _BLK,),
        in_specs=[
            pl.BlockSpec((_BLK, _DIN), lambda i: (i, 0)),
            pl.BlockSpec((_DIN, _H), lambda i: (0, 0)),
            pl.BlockSpec((_DE, _H), lambda i: (0, 0)),
            pl.BlockSpec((_NC, _NS, _BLK), lambda i: (0, 0, i)),
            pl.BlockSpec((_NC, _BLK, _DE), lambda i: (0, i, 0)),
        ],
        out_specs=pl.BlockSpec((_BLK, _H), lambda i: (i, 0)),
        out_shape=jax.ShapeDtypeStruct((_N, _H), jnp.float32),
    )(h_in, w1t, w2t, sums_t, cnts)


def kernel(h_in, edge_index, edge_attr, weights):
    attr4 = (edge_attr.reshape(_NCHUNKS, _CHUNK, 2, 8)
             .transpose(2, 0, 3, 1))
    ei3 = (edge_index.transpose(1, 0)
           .reshape(_NCHUNKS, _CHUNK, 2)
           .transpose(0, 2, 1))
    sums_t, cnts = _sc_segment_sum(ei3, attr4)
    w1t = weights[:, :_DIN].T
    w2t = weights[:, _DIN:].T
    return _tc_combine(h_in, w1t, w2t, sums_t, cnts)


# speedup vs baseline: 1.0049x; 1.0049x over previous
import functools

import jax
import jax.numpy as jnp
from jax import lax
from jax.experimental import pallas as pl
from jax.experimental.pallas import tpu as pltpu
from jax.experimental.pallas import tpu_sc as plsc

_N = 10000
_E = 320000
_DE = 16
_DIN = 128
_H = 128

_CHUNK = 128
_NCHUNKS = _E // _CHUNK
_NC = 2
_NS = 16
_CPT = 8
_HALF = _NCHUNKS // _NC
_TRIPS = _HALF // _CPT
_TAIL = _HALF - _TRIPS * _CPT
_ROWS_PER_TILE = 632
_NPAD = _ROWS_PER_TILE * _NS
_FOLD = 640


def _sc_segment_sum(ei3, attr4):
    mesh = plsc.VectorSubcoreMesh(core_axis_name="c", subcore_axis_name="s")

    @functools.partial(
        pl.kernel,
        mesh=mesh,
        compiler_params=pltpu.CompilerParams(use_tc_tiling_on_sc=False,
                                             needs_layout_passes=False),
        out_type=[
            jax.ShapeDtypeStruct((_NC, _NS, _NPAD), jnp.float32),
            jax.ShapeDtypeStruct((_NC, _NPAD, _DE), jnp.float32),
        ],
        scratch_types=[
            pltpu.VMEM((2, _CPT, 2, _CHUNK), jnp.int32),
            pltpu.VMEM((2, _CPT, _CHUNK), jnp.float32),
            pltpu.VMEM((2, _NPAD), jnp.float32),
            pltpu.VMEM((_ROWS_PER_TILE, _DE), jnp.float32),
            pltpu.VMEM((_FOLD, _DE), jnp.float32),
            pltpu.VMEM((_FOLD, _DE), jnp.float32),
            pltpu.VMEM((5, _CHUNK), jnp.int32),
            pltpu.VMEM_SHARED((_FOLD, _DE), jnp.float32),
            pltpu.SemaphoreType.DMA((2,)),
            pltpu.SemaphoreType.DMA((2,)),
        ],
    )
    def seg(ei_hbm, attr_hbm, sums_hbm, cnts_hbm,
            idx_v, feat_v, plane_v, stage_v, hist_v, fold_v, iden_v,
            acc_fold, isem, asem):
        cid = lax.axis_index("c")
        sid = lax.axis_index("s")
        ones16 = jnp.ones((_DE,), jnp.float32)
        zeros16 = jnp.zeros((_DE,), jnp.float32)

        def zero_plane(i, carry):
            plane_v[0, pl.ds(i * _DE, _DE)] = zeros16
            plane_v[1, pl.ds(i * _DE, _DE)] = zeros16
            return carry
        lax.fori_loop(0, _NPAD // _DE, zero_plane, None)

        def zero_hist(i, carry):
            hist_v[i, :] = zeros16
            return carry
        lax.fori_loop(0, _FOLD, zero_hist, None)

        lane = lax.broadcasted_iota(jnp.int32, (_DE,), 0)
        for c in range(5):
            for k in range(8):
                iden_v[c, pl.ds(k * _DE, _DE)] = lane + (c * _CHUNK + k * _DE)

        pltpu.sync_copy(hist_v.at[pl.ds(sid * (_FOLD // _NS), _FOLD // _NS)],
                        acc_fold.at[pl.ds(sid * (_FOLD // _NS), _FOLD // _NS)])
        plsc.subcore_barrier()

        half0 = cid * _HALF
        fa = sid // 8
        fr = sid % 8

        def loads(t, b, n=_CPT):
            c0 = half0 + t * _CPT
            i_cp = pltpu.make_async_copy(
                ei_hbm.at[pl.ds(c0, n)], idx_v.at[b, pl.ds(0, n)], isem.at[b])
            a_cp = pltpu.make_async_copy(
                attr_hbm.at[fa, pl.ds(c0, n), fr],
                feat_v.at[b, pl.ds(0, n)], asem.at[b])
            return i_cp, a_cp

        def fire(t, b, n=_CPT):
            i_cp, a_cp = loads(t, b, n)
            i_cp.start()
            a_cp.start()

        def consume(t, b, n_chunks):
            i_cp, a_cp = loads(t, b, n_chunks)
            i_cp.wait()
            a_cp.wait()
            for j in range(n_chunks):
                cc = half0 + t * _CPT + j

                ivs = [idx_v[b, j, 1, pl.ds(k * _DE, _DE)]
                       for k in range(8)]
                avs = [feat_v[b, j, pl.ds(k * _DE, _DE)]
                       for k in range(8)]
                for k in range(8):
                    plsc.addupdate_scatter(plane_v.at[k % 2], [ivs[k]],
                                           avs[k])

                @pl.when(cc % _NS == sid)
                def _():
                    for k in range(8):
                        iv = idx_v[b, j, 1, pl.ds(k * _DE, _DE)]
                        plsc.addupdate_scatter(
                            hist_v,
                            [jax.lax.shift_right_logical(iv, 4),
                             jnp.bitwise_and(iv, 15)],
                            ones16)

        fire(0, 0)
        fire(1, 1)

        def body(i, carry):
            for b in range(2):
                t = 2 * i + b
                consume(t, b, _CPT)

                @pl.when(t + 2 < _TRIPS)
                def _():
                    fire(t + 2, b)
            return carry
        lax.fori_loop(0, _TRIPS // 2, body, None)

        fire(_TRIPS, 0, _TAIL)
        consume(_TRIPS, 0, _TAIL)

        def combine(i, carry):
            sl = pl.ds(i * _DE, _DE)
            plane_v[0, sl] = plane_v[0, sl] + plane_v[1, sl]
            return carry
        lax.fori_loop(0, _NPAD // _DE, combine, None)
        pltpu.sync_copy(plane_v.at[0], sums_hbm.at[cid, sid])

        for c in range(5):
            pltpu.sync_copy(hist_v.at[pl.ds(c * _CHUNK, _CHUNK)],
                            acc_fold.at[iden_v.at[c]], add=True)

        plsc.subcore_barrier()

        pltpu.sync_copy(acc_fold, fold_v)
        row0 = sid * _ROWS_PER_TILE

        def unfold(n, carry):
            node = row0 + n
            rr = jnp.full((_DE,), jax.lax.shift_right_logical(node, 4),
                          jnp.int32)
            ll = jnp.full((_DE,), jnp.bitwise_and(node, 15), jnp.int32)
            stage_v[n, :] = plsc.load_gather(fold_v, [rr, ll])
            return carry
        lax.fori_loop(0, _ROWS_PER_TILE, unfold, None)
        pltpu.sync_copy(stage_v, cnts_hbm.at[cid, pl.ds(row0, _ROWS_PER_TILE)])

    return seg(ei3, attr4)


_BLK = 1024


def _tc_body(h_ref, w1_ref, w2_ref, s_ref, c_ref, o_ref):
    st = s_ref[0] + s_ref[1]
    cnt = c_ref[0] + c_ref[1]
    proj = lax.dot_general(st, w2_ref[...], (((0,), (0,)), ((), ())),
                           preferred_element_type=jnp.float32)
    recip = 1.0 / jnp.maximum(cnt[:, 0:1], 1.0)
    o_ref[...] = (
        jnp.dot(h_ref[...], w1_ref[...],
                preferred_element_type=jnp.float32)
        + proj * recip
    )


def _tc_combine(h_in, w1t, w2t, sums_t, cnts):
    return pl.pallas_call(
        _tc_body,
        grid=((_N + _BLK - 1) // _BLK,),
        in_specs=[
            pl.BlockSpec((_BLK, _DIN), lambda i: (i, 0)),
            pl.BlockSpec((_DIN, _H), lambda i: (0, 0)),
            pl.BlockSpec((_DE, _H), lambda i: (0, 0)),
            pl.BlockSpec((_NC, _NS, _BLK), lambda i: (0, 0, i)),
            pl.BlockSpec((_NC, _BLK, _DE), lambda i: (0, i, 0)),
        ],
        out_specs=pl.BlockSpec((_BLK, _H), lambda i: (i, 0)),
        out_shape=jax.ShapeDtypeStruct((_N, _H), jnp.float32),
    )(h_in, w1t, w2t, sums_t, cnts)


def kernel(h_in, edge_index, edge_attr, weights):
    attr4 = (edge_attr.reshape(_NCHUNKS, _CHUNK, 2, 8)
             .transpose(2, 0, 3, 1))
    ei3 = (edge_index.transpose(1, 0)
           .reshape(_NCHUNKS, _CHUNK, 2)
           .transpose(0, 2, 1))
    sums_t, cnts = _sc_segment_sum(ei3, attr4)
    w1t = weights[:, :_DIN].T
    w2t = weights[:, _DIN:].T
    return _tc_combine(h_in, w1t, w2t, sums_t, cnts)
